# baseline (speedup 1.0000x reference)
import functools

import jax
import jax.numpy as jnp
from jax import lax
from jax.experimental import pallas as pl
from jax.experimental.pallas import tpu as pltpu
from jax.experimental.pallas import tpu_sc as plsc

_BATCH = 16384
_D = 64
_CLS = 100000
_SPLIT = 50048
_LO1 = 49920
_WINB_MAIN = 99968 - _LO1
_TAILBASE = 99968
_NTAIL = _CLS - _TAILBASE
_WINB = _WINB_MAIN + 128
_NC = 2
_NS = 16
_NW = _NC * _NS
_DIMS_PW = _D // _NW
_LANES = 16
_FCHUNK = 8192
_NFC = _BATCH // _FCHUNK
_UNROLL = 4

_mesh = plsc.VectorSubcoreMesh(core_axis_name="c", subcore_axis_name="s")


@functools.partial(
    pl.kernel,
    out_type=jax.ShapeDtypeStruct((_NW, _LANES), jnp.float32),
    mesh=_mesh,
    scratch_types=[
        pltpu.VMEM((_SPLIT,), jnp.float32),
        pltpu.VMEM((_WINB,), jnp.float32),
        pltpu.VMEM((_BATCH,), jnp.int32),
        pltpu.VMEM((_FCHUNK,), jnp.float32),
        pltpu.VMEM((_NTAIL,), jnp.float32),
        pltpu.VMEM((_NTAIL,), jnp.float32),
        pltpu.VMEM((_LANES,), jnp.float32),
        pltpu.SemaphoreType.DMA,
        pltpu.SemaphoreType.DMA,
    ],
    compiler_params=pltpu.CompilerParams(needs_layout_passes=False),
)
def _center_loss_partials(feat_hbm, lab_hbm, cent_hbm, tail_hbm, out_hbm,
                          bufa_v, bufb_v, lab_v, fbuf_v, tail0_v, tail1_v,
                          acc_v, sema, semb):
    wid = lax.axis_index("c") * _NS + lax.axis_index("s")
    d0 = wid * _DIMS_PW

    def fire_a(d):
        return pltpu.async_copy(
            cent_hbm.at[d].at[pl.ds(0, _SPLIT)], bufa_v, sema)

    def fire_b(d):
        return pltpu.async_copy(
            cent_hbm.at[d].at[pl.ds(_LO1, _WINB_MAIN)],
            bufb_v.at[pl.ds(0, _WINB_MAIN)], semb)

    wa = fire_a(d0)
    wb = fire_b(d0)
    tails = (tail0_v, tail1_v)
    for t in range(_DIMS_PW):
        pltpu.sync_copy(tail_hbm.at[d0 + t], tails[t])
    pltpu.sync_copy(lab_hbm, lab_v)

    accs = [jnp.zeros((_LANES,), jnp.float32) for _ in range(_UNROLL)]
    for s in range(2 * _DIMS_PW):
        d = d0 + s // 2
        is_b = s % 2
        if is_b:
            wb.wait()
            for k in range(_NTAIL // _LANES):
                bufb_v[pl.ds(_WINB_MAIN + k * _LANES, _LANES)] = (
                    tails[s // 2][pl.ds(k * _LANES, _LANES)])
            buf = bufb_v
        else:
            wa.wait()
            buf = bufa_v
        for fc in range(_NFC):
            pltpu.sync_copy(
                feat_hbm.at[d, pl.ds(fc * _FCHUNK, _FCHUNK)], fbuf_v)
            base = fc * _FCHUNK

            def blk(i, accs, buf=buf, base=base, is_b=is_b):
                accs = list(accs)
                for u in range(_UNROLL):
                    o = (i * _UNROLL + u) * _LANES
                    raw = lab_v[pl.ds(base + o, _LANES)]
                    f = fbuf_v[pl.ds(o, _LANES)]
                    if is_b:
                        m = raw >= _SPLIT
                        c = plsc.load_gather(buf, [raw - _LO1], mask=m)
                    else:
                        m = raw < _SPLIT
                        c = plsc.load_gather(buf, [raw], mask=m)
                    df = jnp.where(m, f - c, 0.0)
                    accs[u] = accs[u] + df * df
                return tuple(accs)

            accs = lax.fori_loop(
                0, _FCHUNK // (_LANES * _UNROLL), blk, tuple(accs))
            accs = list(accs)
        if s // 2 + 1 < _DIMS_PW:
            if is_b:
                wb = fire_b(d + 1)
            else:
                wa = fire_a(d + 1)

    acc_v[...] = (accs[0] + accs[1]) + (accs[2] + accs[3])
    pltpu.sync_copy(acc_v, out_hbm.at[wid])


def kernel(features, labels, centers):
    labels = labels.astype(jnp.int32)
    cent_t = centers.T
    tail = lax.slice(cent_t, (0, _TAILBASE), (_D, _CLS))
    partials = _center_loss_partials(features.T, labels, cent_t, tail)
    return jnp.sum(partials) / jnp.float32(_BATCH)


# speedup vs baseline: 1.1459x; 1.1459x over previous
import functools

import jax
import jax.numpy as jnp
from jax import lax
from jax.experimental import pallas as pl
from jax.experimental.pallas import tpu as pltpu
from jax.experimental.pallas import tpu_sc as plsc

_BATCH = 16384
_D = 64
_CLS = 100000
_NC = 2
_NS = 16
_NW = _NC * _NS
_WAVES = _D // _NW
_LANES = 16
_FCHUNK = 4096
_NFC = _BATCH // _FCHUNK
_UNROLL = 8

_mesh = plsc.VectorSubcoreMesh(core_axis_name="c", subcore_axis_name="s")


@functools.partial(
    pl.kernel,
    out_type=jax.ShapeDtypeStruct((_NW, _LANES), jnp.float32),
    mesh=_mesh,
    scratch_types=[
        pltpu.VMEM((_CLS,), jnp.float32),
        pltpu.VMEM((_BATCH,), jnp.int32),
        pltpu.VMEM((_FCHUNK,), jnp.float32),
        pltpu.VMEM((_FCHUNK,), jnp.float32),
        pltpu.VMEM((_LANES,), jnp.float32),
        pltpu.SemaphoreType.DMA,
        pltpu.SemaphoreType.DMA,
        pltpu.SemaphoreType.DMA,
    ],
    compiler_params=pltpu.CompilerParams(needs_layout_passes=False),
)
def _center_loss_partials(feat_hbm, lab_hbm, cent_hbm, out_hbm,
                          crow_v, lab_v, fb0_v, fb1_v, acc_v,
                          csem, fsem0, fsem1):
    wid = lax.axis_index("c") * _NS + lax.axis_index("s")

    fbufs = (fb0_v, fb1_v)
    fsems = (fsem0, fsem1)

    def fire_fchunk(d, fc):
        return pltpu.async_copy(
            feat_hbm.at[d, pl.ds(fc * _FCHUNK, _FCHUNK)],
            fbufs[fc % 2], fsems[fc % 2])

    d0 = wid * _WAVES
    cw = pltpu.async_copy(cent_hbm.at[d0], crow_v, csem)
    fw = fire_fchunk(d0, 0)
    pltpu.sync_copy(lab_hbm, lab_v)

    accs = [jnp.zeros((_LANES,), jnp.float32) for _ in range(_UNROLL)]
    for w in range(_WAVES):
        d = d0 + w
        cw.wait()
        for fc in range(_NFC):
            fw.wait()
            fbuf = fbufs[fc % 2]
            if fc + 1 < _NFC:
                fw = fire_fchunk(d, fc + 1)
            base = fc * _FCHUNK

            def blk(i, accs, fbuf=fbuf, base=base):
                accs = list(accs)
                for u in range(_UNROLL):
                    o = (i * _UNROLL + u) * _LANES
                    idx = lab_v[pl.ds(base + o, _LANES)]
                    c = plsc.load_gather(crow_v, [idx])
                    f = fbuf[pl.ds(o, _LANES)]
                    df = f - c
                    accs[u] = accs[u] + df * df
                return tuple(accs)

            accs = lax.fori_loop(
                0, _FCHUNK // (_LANES * _UNROLL), blk, tuple(accs))
            accs = list(accs)
        if w + 1 < _WAVES:
            cw = pltpu.async_copy(cent_hbm.at[d0 + w + 1], crow_v, csem)
            fw = fire_fchunk(d0 + w + 1, 0)

    r = accs[0]
    for u in range(1, _UNROLL):
        r = r + accs[u]
    acc_v[...] = r
    pltpu.sync_copy(acc_v, out_hbm.at[wid])


def kernel(features, labels, centers):
    labels = labels.astype(jnp.int32)
    partials = _center_loss_partials(features.T, labels, centers.T)
    return jnp.sum(partials) / jnp.float32(_BATCH)
